# Initial kernel scaffold; baseline (speedup 1.0000x reference)
import jax
import jax.numpy as jnp
from jax.experimental import pallas as pl


def kernel(x, edge_index, W1, b1, W2, b2):
    raise NotImplementedError("write your pallas kernel here")


# speedup vs baseline: 10.9151x; 10.9151x over previous
import functools

import jax
import jax.numpy as jnp
from jax import lax
from jax.experimental import pallas as pl
from jax.experimental.pallas import tpu as pltpu
from jax.experimental.pallas import tpu_sc as plsc

NC, NS = 2, 16
CHUNKS, CW = 80, 125
NP = 10240
RB = 1024


def _sc_mesh():
    return plsc.VectorSubcoreMesh(core_axis_name="c", subcore_axis_name="s")


def _degrees(edges_t, ones_np):
    @functools.partial(
        pl.kernel,
        out_type=jax.ShapeDtypeStruct((2 * NP,), jnp.float32),
        mesh=_sc_mesh(),
        scratch_types=[
            pltpu.VMEM((CHUNKS, CW), jnp.int32),
            pltpu.VMEM((CW,), jnp.float32),
            pltpu.VMEM_SHARED((NP,), jnp.float32),
        ],
    )
    def k(edges_hbm, ones_hbm, deg_hbm, idx_v, ones_v, deg_sh):
        c = lax.axis_index("c")
        s = lax.axis_index("s")
        pltpu.sync_copy(edges_hbm.at[c * NS + s], idx_v)
        pltpu.sync_copy(ones_hbm.at[pl.ds(0, CW)], ones_v)

        @pl.when(s == 0)
        def _():
            pltpu.sync_copy(ones_hbm, deg_sh)

        plsc.subcore_barrier()

        @pl.loop(0, CHUNKS)
        def _(j):
            pltpu.sync_copy(ones_v, deg_sh.at[idx_v.at[j]], add=True)

        plsc.subcore_barrier()

        @pl.when(s == 0)
        def _():
            pltpu.sync_copy(deg_sh, deg_hbm.at[pl.ds(c * NP, NP)])

    return k(edges_t, ones_np)


def _scatter_rows(vals_flat, src2_t, dst_t, d):
    rows_per = NP // NS

    @functools.partial(
        pl.kernel,
        out_type=jax.ShapeDtypeStruct((2 * NP, d), jnp.float32),
        mesh=_sc_mesh(),
        scratch_types=[
            pltpu.VMEM((CHUNKS, CW), jnp.int32),
            pltpu.VMEM((CHUNKS, CW), jnp.int32),
            pltpu.VMEM((CW, d), jnp.float32),
            pltpu.SemaphoreType.DMA,
            pltpu.VMEM_SHARED((NP, d), jnp.float32),
        ],
    )
    def k(vals_hbm, src_hbm, dst_hbm, out_hbm, src_v, dst_v, rows_v, sem, acc_sh):
        c = lax.axis_index("c")
        s = lax.axis_index("s")
        base = s * rows_per
        pltpu.sync_copy(vals_hbm.at[pl.ds(c * NP + base, rows_per)],
                        acc_sh.at[pl.ds(base, rows_per)])
        pltpu.sync_copy(src_hbm.at[c * NS + s], src_v)
        pltpu.sync_copy(dst_hbm.at[s], dst_v)
        plsc.subcore_barrier()

        @pl.loop(0, CHUNKS)
        def _(j):
            pltpu.async_copy(vals_hbm.at[src_v.at[j]], rows_v, sem).wait()
            pltpu.sync_copy(rows_v, acc_sh.at[dst_v.at[j]], add=True)

        plsc.subcore_barrier()
        pltpu.sync_copy(acc_sh.at[pl.ds(base, rows_per)],
                        out_hbm.at[pl.ds(c * NP + base, rows_per)])

    return k(vals_flat, src2_t, dst_t)


def _prep(x, deg_o):
    n, d = x.shape
    half = d // 2

    def body(x_ref, dg_ref, out_ref):
        do = lax.rsqrt(dg_ref[...])
        xs = x_ref[...] * do
        out_ref[0] = xs[:, :half]
        out_ref[1] = xs[:, half:]

    return pl.pallas_call(
        body,
        grid=(n // RB,),
        in_specs=[pl.BlockSpec((RB, d), lambda i: (i, 0)),
                  pl.BlockSpec((RB, 1), lambda i: (i, 0))],
        out_specs=pl.BlockSpec((2, RB, half), lambda i: (0, i, 0)),
        out_shape=jax.ShapeDtypeStruct((2, n, half), jnp.float32),
    )(x, deg_o)


def _mm(agg2, deg_i, deg_o, W1, b1, W2):
    _, n, half = agg2.shape
    d_hid = W1.shape[1]

    def body(a_ref, di_ref, do_ref, w1_ref, b1_ref, w2_ref, out_ref):
        di = lax.rsqrt(di_ref[...])
        do = lax.rsqrt(do_ref[...])
        agg = jnp.concatenate([a_ref[0], a_ref[1]], axis=1)
        h1 = jnp.dot(agg * di, w1_ref[...], preferred_element_type=jnp.float32)
        h1 = jnp.maximum(h1 + b1_ref[...], 0.0)
        t = jnp.dot(h1 * do, w2_ref[...], preferred_element_type=jnp.float32)
        out_ref[0] = t[:, :half]
        out_ref[1] = t[:, half:]

    return pl.pallas_call(
        body,
        grid=(n // RB,),
        in_specs=[pl.BlockSpec((2, RB, half), lambda i: (0, i, 0)),
                  pl.BlockSpec((RB, 1), lambda i: (i, 0)),
                  pl.BlockSpec((RB, 1), lambda i: (i, 0)),
                  pl.BlockSpec(W1.shape, lambda i: (0, 0)),
                  pl.BlockSpec((1, d_hid), lambda i: (0, 0)),
                  pl.BlockSpec(W2.shape, lambda i: (0, 0))],
        out_specs=pl.BlockSpec((2, RB, half), lambda i: (0, i, 0)),
        out_shape=jax.ShapeDtypeStruct((2, n, half), jnp.float32),
    )(agg2, deg_i, deg_o, W1, b1, W2)


def _final(agg2, deg_i, b2, n_valid):
    _, n, half = agg2.shape
    d = 2 * half

    def body(a_ref, di_ref, b2_ref, out_ref):
        i = pl.program_id(0)
        di = lax.rsqrt(di_ref[...])
        agg = jnp.concatenate([a_ref[0], a_ref[1]], axis=1)
        h2 = jnp.maximum(agg * di + b2_ref[...], 0.0)
        row = i * RB + lax.broadcasted_iota(jnp.int32, (RB, 1), 0)
        h2 = jnp.where(row < n_valid, h2, 0.0)
        part = jnp.sum(h2, axis=0, keepdims=True)

        @pl.when(i == 0)
        def _():
            out_ref[...] = jnp.zeros((1, d), jnp.float32)

        out_ref[...] += part

        @pl.when(i == pl.num_programs(0) - 1)
        def _():
            out_ref[...] = out_ref[...] * (1.0 / n_valid)

    return pl.pallas_call(
        body,
        grid=(n // RB,),
        in_specs=[pl.BlockSpec((2, RB, half), lambda i: (0, i, 0)),
                  pl.BlockSpec((RB, 1), lambda i: (i, 0)),
                  pl.BlockSpec((1, d), lambda i: (0, 0))],
        out_specs=pl.BlockSpec((1, d), lambda i: (0, 0)),
        out_shape=jax.ShapeDtypeStruct((1, d), jnp.float32),
    )(agg2, deg_i, b2)


def kernel(x, edge_index, W1, b1, W2, b2):
    n, d_in = x.shape
    half = d_in // 2

    src = edge_index[0]
    dst = edge_index[1]
    edges_t = edge_index.reshape(2 * NS, CHUNKS, CW)
    ones_np = jnp.ones((NP,), jnp.float32)

    deg = _degrees(edges_t, ones_np)
    deg_o = deg[:NP].reshape(NP, 1)
    deg_i = deg[NP:].reshape(NP, 1)

    src2 = jnp.stack([src, src + NP]).reshape(2 * NS, CHUNKS, CW)
    dst_t = dst.reshape(NS, CHUNKS, CW)

    x_pad = jnp.pad(x, ((0, NP - n), (0, 0)))
    xs2 = _prep(x_pad, deg_o)
    agg1 = _scatter_rows(xs2.reshape(2 * NP, half), src2, dst_t, half)
    t2 = _mm(agg1.reshape(2, NP, half), deg_i, deg_o, W1,
             b1.reshape(1, -1), W2)
    agg2 = _scatter_rows(t2.reshape(2 * NP, half), src2, dst_t, half)
    return _final(agg2.reshape(2, NP, half), deg_i, b2.reshape(1, -1), n)
